# baseline (speedup 1.0000x reference)
import functools

import numpy as np
import jax
import jax.numpy as jnp
from jax import lax
from jax.experimental import pallas as pl
from jax.experimental.pallas import tpu as pltpu
from jax.experimental.pallas import tpu_sc as plsc

B = 16384
D = 128
VOCABS = (4, 2, 2, 5, 3, 4, 4)
NF = len(VOCABS)
TOT = int(np.prod(VOCABS))
ROWS = sum(VOCABS)
RPAD = 32

STRIDES = tuple(int(np.prod(VOCABS[i + 1:])) for i in range(NF))
OFFSETS = tuple(int(sum(VOCABS[:i])) for i in range(NF))


def _build_onehot_t() -> np.ndarray:
    n = np.arange(TOT)
    u = np.zeros((RPAD, TOT), np.float32)
    for i in range(NF):
        c = (n // STRIDES[i]) % VOCABS[i]
        u[OFFSETS[i] + c, n] = 1.0
    return u


_UT = _build_onehot_t()


def _lut_body(u_ref, idx_ref, w_ref, t_ref, flat_ref):
    u = u_ref[...]
    w = w_ref[...]
    s = lax.dot_general(w, u, (((0,), (0,)), ((), ())),
                        preferred_element_type=jnp.float32,
                        precision=lax.Precision.HIGHEST)
    q = jnp.sum(w * w, axis=1, keepdims=True)
    t = jnp.sum(s * s, axis=0, keepdims=True)
    t = t - lax.dot_general(q, u, (((0,), (0,)), ((), ())),
                            preferred_element_type=jnp.float32,
                            precision=lax.Precision.HIGHEST)
    t_ref[...] = jnp.squeeze(t, axis=0)
    idx = idx_ref[...]
    flat = idx[0] * STRIDES[0]
    for i in range(1, NF):
        flat = flat + idx[i] * STRIDES[i]
    flat_ref[...] = flat


def _build_lut_and_flat(idx, wcat):
    return pl.pallas_call(
        _lut_body,
        out_shape=(
            jax.ShapeDtypeStruct((TOT,), jnp.float32),
            jax.ShapeDtypeStruct((B,), jnp.int32),
        ),
    )(_UT, idx, wcat)


_NC = 2
_NS = 16
_NW = _NC * _NS
BPW = B // _NW
_L = 16


@functools.cache
def _make_fm_gather():
    mesh = plsc.VectorSubcoreMesh(
        core_axis_name="c", subcore_axis_name="s", num_cores=_NC, num_subcores=_NS
    )

    @functools.partial(
        pl.kernel,
        out_type=jax.ShapeDtypeStruct((B,), jnp.float32),
        mesh=mesh,
        compiler_params=pltpu.CompilerParams(needs_layout_passes=False),
        scratch_types=[
            pltpu.VMEM((TOT,), jnp.float32),
            pltpu.VMEM((BPW,), jnp.int32),
            pltpu.VMEM((BPW,), jnp.float32),
            pltpu.SemaphoreType.DMA,
            pltpu.SemaphoreType.DMA,
        ],
    )
    def _fm_gather(flat_hbm, lut_hbm, out_hbm, lut_v, flat_v, out_v, s_lut, s_idx):
        wid = lax.axis_index("s") * _NC + lax.axis_index("c")
        base = wid * BPW
        lut_cp = pltpu.make_async_copy(lut_hbm, lut_v, s_lut)
        flat_cp = pltpu.make_async_copy(
            flat_hbm.at[pl.ds(base, BPW)], flat_v, s_idx)
        lut_cp.start()
        flat_cp.start()
        flat_cp.wait()
        lut_cp.wait()
        for j in range(BPW // _L):
            f = flat_v[pl.ds(j * _L, _L)]
            out_v[pl.ds(j * _L, _L)] = plsc.load_gather(lut_v, [f])
        pltpu.sync_copy(out_v, out_hbm.at[pl.ds(base, BPW)])

    return _fm_gather


def kernel(input, W1, W2, W3, W4, W5, W6, W7):
    idx = input.astype(jnp.int32)
    wcat = jnp.concatenate([W1, W2, W3, W4, W5, W6, W7], axis=0)
    wcat = jnp.pad(wcat, ((0, RPAD - ROWS), (0, 0)))
    lut, flat = _build_lut_and_flat(idx, wcat)
    out = _make_fm_gather()(flat, lut)
    return out.reshape(B, 1)


# speedup vs baseline: 1.1749x; 1.1749x over previous
import functools

import numpy as np
import jax
import jax.numpy as jnp
from jax import lax
from jax.experimental import pallas as pl
from jax.experimental.pallas import tpu as pltpu
from jax.experimental.pallas import tpu_sc as plsc

B = 16384
D = 128
VOCABS = (4, 2, 2, 5, 3, 4, 4)
NF = len(VOCABS)
TOT = int(np.prod(VOCABS))
ROWS = sum(VOCABS)

STRIDES = tuple(int(np.prod(VOCABS[i + 1:])) for i in range(NF))
OFFSETS = tuple(int(sum(VOCABS[:i])) for i in range(NF))


def _build_onehot_t() -> np.ndarray:
    n = np.arange(TOT)
    u = np.zeros((ROWS, TOT), np.float32)
    for i in range(NF):
        c = (n // STRIDES[i]) % VOCABS[i]
        u[OFFSETS[i] + c, n] = 1.0
    return u


_UT = _build_onehot_t()


def _lut_body(u_ref, idx_ref, *rest):
    (*w_refs, t_ref, flat_ref) = rest
    u = u_ref[...]
    w = jnp.concatenate([r[...] for r in w_refs], axis=0)
    s = lax.dot_general(w, u, (((0,), (0,)), ((), ())),
                        preferred_element_type=jnp.float32,
                        precision=lax.Precision.HIGHEST)
    q = jnp.sum(w * w, axis=1, keepdims=True)
    t = jnp.sum(s * s, axis=0, keepdims=True)
    t = t - lax.dot_general(q, u, (((0,), (0,)), ((), ())),
                            preferred_element_type=jnp.float32,
                            precision=lax.Precision.HIGHEST)
    t_ref[...] = jnp.squeeze(t, axis=0)
    idx = idx_ref[...]
    flat = idx[0] * STRIDES[0]
    for i in range(1, NF):
        flat = flat + idx[i] * STRIDES[i]
    flat_ref[...] = flat


def _build_lut_and_flat(idx, *ws):
    return pl.pallas_call(
        _lut_body,
        out_shape=(
            jax.ShapeDtypeStruct((TOT,), jnp.float32),
            jax.ShapeDtypeStruct((B,), jnp.int32),
        ),
    )(_UT, idx, *ws)


_NC = 2
_NS = 16
_NW = _NC * _NS
BPW = B // _NW
_L = 16


@functools.cache
def _make_fm_gather():
    mesh = plsc.VectorSubcoreMesh(
        core_axis_name="c", subcore_axis_name="s", num_cores=_NC, num_subcores=_NS
    )

    @functools.partial(
        pl.kernel,
        out_type=jax.ShapeDtypeStruct((B,), jnp.float32),
        mesh=mesh,
        compiler_params=pltpu.CompilerParams(needs_layout_passes=False),
        scratch_types=[
            pltpu.VMEM((TOT,), jnp.float32),
            pltpu.VMEM((BPW,), jnp.int32),
            pltpu.VMEM((BPW,), jnp.float32),
            pltpu.SemaphoreType.DMA,
            pltpu.SemaphoreType.DMA,
        ],
    )
    def _fm_gather(flat_hbm, lut_hbm, out_hbm, lut_v, flat_v, out_v, s_lut, s_idx):
        wid = lax.axis_index("s") * _NC + lax.axis_index("c")
        base = wid * BPW
        lut_cp = pltpu.make_async_copy(lut_hbm, lut_v, s_lut)
        flat_cp = pltpu.make_async_copy(
            flat_hbm.at[pl.ds(base, BPW)], flat_v, s_idx)
        lut_cp.start()
        flat_cp.start()
        flat_cp.wait()
        lut_cp.wait()
        for j in range(BPW // _L):
            f = flat_v[pl.ds(j * _L, _L)]
            out_v[pl.ds(j * _L, _L)] = plsc.load_gather(lut_v, [f])
        pltpu.sync_copy(out_v, out_hbm.at[pl.ds(base, BPW)])

    return _fm_gather


def kernel(input, W1, W2, W3, W4, W5, W6, W7):
    idx = input.astype(jnp.int32)
    lut, flat = _build_lut_and_flat(idx, W1, W2, W3, W4, W5, W6, W7)
    out = _make_fm_gather()(flat, lut)
    return out.reshape(B, 1)


# speedup vs baseline: 1.2787x; 1.0884x over previous
import functools

import numpy as np
import jax
import jax.numpy as jnp
from jax import lax
from jax.experimental import pallas as pl
from jax.experimental.pallas import tpu as pltpu
from jax.experimental.pallas import tpu_sc as plsc

B = 16384
D = 128
VOCABS = (4, 2, 2, 5, 3, 4, 4)
NF = len(VOCABS)
TOT = int(np.prod(VOCABS))
ROWS = sum(VOCABS)

STRIDES = tuple(int(np.prod(VOCABS[i + 1:])) for i in range(NF))
OFFSETS = tuple(int(sum(VOCABS[:i])) for i in range(NF))


def _build_onehot_t() -> np.ndarray:
    n = np.arange(TOT)
    u = np.zeros((ROWS, TOT), np.float32)
    for i in range(NF):
        c = (n // STRIDES[i]) % VOCABS[i]
        u[OFFSETS[i] + c, n] = 1.0
    return u


_UT = _build_onehot_t()


def _lut_body(u_ref, idx_ref, *rest):
    (*w_refs, t_ref, flat_ref) = rest
    u = u_ref[...]
    w = jnp.concatenate([r[...] for r in w_refs], axis=0)
    s = lax.dot_general(w, u, (((0,), (0,)), ((), ())),
                        preferred_element_type=jnp.float32,
                        precision=lax.Precision.HIGHEST)
    q = jnp.sum(w * w, axis=1, keepdims=True)
    t = jnp.sum(s * s, axis=0, keepdims=True)
    t = t - lax.dot_general(q, u, (((0,), (0,)), ((), ())),
                            preferred_element_type=jnp.float32,
                            precision=lax.Precision.HIGHEST)
    t_ref[...] = jnp.squeeze(t, axis=0)
    idx = idx_ref[...]
    flat = idx[0] * STRIDES[0]
    for i in range(1, NF):
        flat = flat + idx[i] * STRIDES[i]
    flat_ref[...] = flat


def _build_lut_and_flat(idx, *ws):
    return pl.pallas_call(
        _lut_body,
        out_shape=(
            jax.ShapeDtypeStruct((TOT,), jnp.float32),
            jax.ShapeDtypeStruct((B,), jnp.int32),
        ),
    )(_UT, idx, *ws)


_NC = 1
_NS = 16
_NW = _NC * _NS
BPW = B // _NW
_L = 16


@functools.cache
def _make_fm_gather():
    mesh = plsc.VectorSubcoreMesh(
        core_axis_name="c", subcore_axis_name="s", num_cores=_NC, num_subcores=_NS
    )

    @functools.partial(
        pl.kernel,
        out_type=jax.ShapeDtypeStruct((B,), jnp.float32),
        mesh=mesh,
        compiler_params=pltpu.CompilerParams(needs_layout_passes=False),
        scratch_types=[
            pltpu.VMEM((TOT,), jnp.float32),
            pltpu.VMEM((BPW,), jnp.int32),
            pltpu.VMEM((BPW,), jnp.float32),
            pltpu.SemaphoreType.DMA,
            pltpu.SemaphoreType.DMA,
        ],
    )
    def _fm_gather(flat_hbm, lut_hbm, out_hbm, lut_v, flat_v, out_v, s_lut, s_idx):
        wid = lax.axis_index("s") * _NC + lax.axis_index("c")
        base = wid * BPW
        lut_cp = pltpu.make_async_copy(lut_hbm, lut_v, s_lut)
        flat_cp = pltpu.make_async_copy(
            flat_hbm.at[pl.ds(base, BPW)], flat_v, s_idx)
        lut_cp.start()
        flat_cp.start()
        flat_cp.wait()
        lut_cp.wait()
        for j in range(BPW // _L):
            f = flat_v[pl.ds(j * _L, _L)]
            out_v[pl.ds(j * _L, _L)] = plsc.load_gather(lut_v, [f])
        pltpu.sync_copy(out_v, out_hbm.at[pl.ds(base, BPW)])

    return _fm_gather


def kernel(input, W1, W2, W3, W4, W5, W6, W7):
    idx = input.astype(jnp.int32)
    lut, flat = _build_lut_and_flat(idx, W1, W2, W3, W4, W5, W6, W7)
    out = _make_fm_gather()(flat, lut)
    return out.reshape(B, 1)


# speedup vs baseline: 1.3598x; 1.0634x over previous
import functools

import numpy as np
import jax
import jax.numpy as jnp
from jax import lax
from jax.experimental import pallas as pl
from jax.experimental.pallas import tpu as pltpu
from jax.experimental.pallas import tpu_sc as plsc

B = 16384
D = 128
VOCABS = (4, 2, 2, 5, 3, 4, 4)
NF = len(VOCABS)
TOT = int(np.prod(VOCABS))
ROWS = sum(VOCABS)

STRIDES = tuple(int(np.prod(VOCABS[i + 1:])) for i in range(NF))
OFFSETS = tuple(int(sum(VOCABS[:i])) for i in range(NF))


def _build_onehot_t() -> np.ndarray:
    n = np.arange(TOT)
    u = np.zeros((ROWS, TOT), np.float32)
    for i in range(NF):
        c = (n // STRIDES[i]) % VOCABS[i]
        u[OFFSETS[i] + c, n] = 1.0
    return u


_UT = _build_onehot_t()


def _lut_body(u_ref, idx_ref, *rest):
    (*w_refs, t_ref, flat_ref) = rest
    u = u_ref[...]
    w = jnp.concatenate([r[...] for r in w_refs], axis=0)
    u16 = u.astype(jnp.bfloat16)
    def _split_dot(a):
        hi = a.astype(jnp.bfloat16)
        lo = (a - hi.astype(jnp.float32)).astype(jnp.bfloat16)
        d = lambda x: lax.dot_general(x, u16, (((0,), (0,)), ((), ())),
                                      preferred_element_type=jnp.float32)
        return d(hi) + d(lo)
    s = _split_dot(w)
    q = jnp.sum(w * w, axis=1, keepdims=True)
    t = jnp.sum(s * s, axis=0, keepdims=True)
    t = t - _split_dot(q)
    t_ref[...] = jnp.squeeze(t, axis=0)
    idx = idx_ref[...]
    flat = idx[0] * STRIDES[0]
    for i in range(1, NF):
        flat = flat + idx[i] * STRIDES[i]
    flat_ref[...] = flat


def _build_lut_and_flat(idx, *ws):
    return pl.pallas_call(
        _lut_body,
        out_shape=(
            jax.ShapeDtypeStruct((TOT,), jnp.float32),
            jax.ShapeDtypeStruct((B,), jnp.int32),
        ),
    )(_UT, idx, *ws)


_NC = 1
_NS = 16
_NW = _NC * _NS
BPW = B // _NW
_L = 16


@functools.cache
def _make_fm_gather():
    mesh = plsc.VectorSubcoreMesh(
        core_axis_name="c", subcore_axis_name="s", num_cores=_NC, num_subcores=_NS
    )

    @functools.partial(
        pl.kernel,
        out_type=jax.ShapeDtypeStruct((B,), jnp.float32),
        mesh=mesh,
        compiler_params=pltpu.CompilerParams(needs_layout_passes=False),
        scratch_types=[
            pltpu.VMEM((TOT,), jnp.float32),
            pltpu.VMEM((BPW,), jnp.int32),
            pltpu.VMEM((BPW,), jnp.float32),
            pltpu.SemaphoreType.DMA,
            pltpu.SemaphoreType.DMA,
        ],
    )
    def _fm_gather(flat_hbm, lut_hbm, out_hbm, lut_v, flat_v, out_v, s_lut, s_idx):
        wid = lax.axis_index("s") * _NC + lax.axis_index("c")
        base = wid * BPW
        lut_cp = pltpu.make_async_copy(lut_hbm, lut_v, s_lut)
        flat_cp = pltpu.make_async_copy(
            flat_hbm.at[pl.ds(base, BPW)], flat_v, s_idx)
        lut_cp.start()
        flat_cp.start()
        flat_cp.wait()
        lut_cp.wait()
        for j in range(BPW // _L):
            f = flat_v[pl.ds(j * _L, _L)]
            out_v[pl.ds(j * _L, _L)] = plsc.load_gather(lut_v, [f])
        pltpu.sync_copy(out_v, out_hbm.at[pl.ds(base, BPW)])

    return _fm_gather


def kernel(input, W1, W2, W3, W4, W5, W6, W7):
    idx = input.astype(jnp.int32)
    lut, flat = _build_lut_and_flat(idx, W1, W2, W3, W4, W5, W6, W7)
    out = _make_fm_gather()(flat, lut)
    return out.reshape(B, 1)


# speedup vs baseline: 1.3671x; 1.0053x over previous
import functools

import numpy as np
import jax
import jax.numpy as jnp
from jax import lax
from jax.experimental import pallas as pl
from jax.experimental.pallas import tpu as pltpu
from jax.experimental.pallas import tpu_sc as plsc

B = 16384
D = 128
VOCABS = (4, 2, 2, 5, 3, 4, 4)
NF = len(VOCABS)
TOT = int(np.prod(VOCABS))
ROWS = sum(VOCABS)

STRIDES = tuple(int(np.prod(VOCABS[i + 1:])) for i in range(NF))
OFFSETS = tuple(int(sum(VOCABS[:i])) for i in range(NF))


def _build_onehot_t() -> np.ndarray:
    n = np.arange(TOT)
    u = np.zeros((ROWS, TOT), np.float32)
    for i in range(NF):
        c = (n // STRIDES[i]) % VOCABS[i]
        u[OFFSETS[i] + c, n] = 1.0
    return u.astype(jnp.bfloat16)


_UT = _build_onehot_t()


def _lut_body(u_ref, idx_ref, *rest):
    (*w_refs, t_ref, flat_ref) = rest
    u16 = u_ref[...]
    w = jnp.concatenate([r[...] for r in w_refs], axis=0)
    def _split_dot(a):
        hi = a.astype(jnp.bfloat16)
        lo = (a - hi.astype(jnp.float32)).astype(jnp.bfloat16)
        d = lambda x: lax.dot_general(x, u16, (((0,), (0,)), ((), ())),
                                      preferred_element_type=jnp.float32)
        return d(hi) + d(lo)
    s = _split_dot(w)
    q = jnp.sum(w * w, axis=1, keepdims=True)
    t = jnp.sum(s * s, axis=0, keepdims=True)
    t = t - _split_dot(q)
    t_ref[...] = jnp.squeeze(t, axis=0)
    idx = idx_ref[...]
    flat = idx[0] * STRIDES[0]
    for i in range(1, NF):
        flat = flat + idx[i] * STRIDES[i]
    flat_ref[...] = flat


def _build_lut_and_flat(idx, *ws):
    return pl.pallas_call(
        _lut_body,
        out_shape=(
            jax.ShapeDtypeStruct((TOT,), jnp.float32),
            jax.ShapeDtypeStruct((B,), jnp.int32),
        ),
    )(_UT, idx, *ws)


_NC = 1
_NS = 16
_NW = _NC * _NS
BPW = B // _NW
_L = 16


@functools.cache
def _make_fm_gather():
    mesh = plsc.VectorSubcoreMesh(
        core_axis_name="c", subcore_axis_name="s", num_cores=_NC, num_subcores=_NS
    )

    @functools.partial(
        pl.kernel,
        out_type=jax.ShapeDtypeStruct((B,), jnp.float32),
        mesh=mesh,
        compiler_params=pltpu.CompilerParams(needs_layout_passes=False),
        scratch_types=[
            pltpu.VMEM((TOT,), jnp.float32),
            pltpu.VMEM((BPW,), jnp.int32),
            pltpu.VMEM((BPW,), jnp.float32),
            pltpu.SemaphoreType.DMA,
            pltpu.SemaphoreType.DMA,
        ],
    )
    def _fm_gather(flat_hbm, lut_hbm, out_hbm, lut_v, flat_v, out_v, s_lut, s_idx):
        wid = lax.axis_index("s") * _NC + lax.axis_index("c")
        base = wid * BPW
        lut_cp = pltpu.make_async_copy(lut_hbm, lut_v, s_lut)
        flat_cp = pltpu.make_async_copy(
            flat_hbm.at[pl.ds(base, BPW)], flat_v, s_idx)
        lut_cp.start()
        flat_cp.start()
        flat_cp.wait()
        lut_cp.wait()
        for j in range(BPW // _L):
            f = flat_v[pl.ds(j * _L, _L)]
            out_v[pl.ds(j * _L, _L)] = plsc.load_gather(lut_v, [f])
        pltpu.sync_copy(out_v, out_hbm.at[pl.ds(base, BPW)])

    return _fm_gather


def kernel(input, W1, W2, W3, W4, W5, W6, W7):
    idx = input.astype(jnp.int32)
    lut, flat = _build_lut_and_flat(idx, W1, W2, W3, W4, W5, W6, W7)
    out = _make_fm_gather()(flat, lut)
    return out.reshape(B, 1)
